# baseline (speedup 1.0000x reference)
import jax
import jax.numpy as jnp
from jax import lax
from jax.experimental import pallas as pl
from jax.experimental.pallas import tpu as pltpu
from jax.experimental.pallas import tpu_sc as plsc

BATCH = 16384
EMBED_DIM = 2
NUM_WORKERS = 32
BPW = BATCH // NUM_WORKERS
OPW = BPW * EMBED_DIM
L = 16


def _sc_body(idx_hbm, tab_hbm, out_hbm, idx_v, tab_v, out_v, sem_i, sem_t):
    c = lax.axis_index("c")
    s = lax.axis_index("s")
    wid = s * 2 + c
    base = wid * BPW
    cp_idx = pltpu.async_copy(idx_hbm.at[pl.ds(base, BPW)], idx_v, sem_i)
    cp_tab = pltpu.async_copy(tab_hbm, tab_v, sem_t)
    cp_idx.wait()
    cp_tab.wait()
    iota = lax.iota(jnp.int32, L)
    two_iota = iota * 2
    col0 = jnp.zeros((L,), jnp.int32)
    col1 = jnp.ones((L,), jnp.int32)
    for k in range(BPW // L):
        idx16 = idx_v[pl.ds(k * L, L)]
        g0 = plsc.load_gather(tab_v, [idx16, col0])
        g1 = plsc.load_gather(tab_v, [idx16, col1])
        plsc.store_scatter(out_v, [two_iota + k * 2 * L], g0)
        plsc.store_scatter(out_v, [two_iota + (k * 2 * L + 1)], g1)
    pltpu.sync_copy(out_v, out_hbm.at[pl.ds(base * EMBED_DIM, OPW)])


def kernel(x, emb_table):
    xi = x.astype(jnp.int32)
    mesh = plsc.VectorSubcoreMesh(core_axis_name="c", subcore_axis_name="s")
    out_flat = pl.kernel(
        _sc_body,
        out_type=jax.ShapeDtypeStruct((BATCH * EMBED_DIM,), jnp.float32),
        mesh=mesh,
        compiler_params=pltpu.CompilerParams(needs_layout_passes=False),
        scratch_types=[
            pltpu.VMEM((BPW,), jnp.int32),
            pltpu.VMEM((3, EMBED_DIM), jnp.float32),
            pltpu.VMEM((OPW,), jnp.float32),
            pltpu.SemaphoreType.DMA,
            pltpu.SemaphoreType.DMA,
        ],
    )(xi, emb_table)
    return out_flat.reshape(BATCH, EMBED_DIM)


# speedup vs baseline: 1.0015x; 1.0015x over previous
import jax
import jax.numpy as jnp
from jax import lax
from jax.experimental import pallas as pl
from jax.experimental.pallas import tpu as pltpu
from jax.experimental.pallas import tpu_sc as plsc

BATCH = 16384
EMBED_DIM = 2
NUM_WORKERS = 32
BPW = BATCH // NUM_WORKERS
OPW = BPW * EMBED_DIM
L = 16


HALF = BPW // 2


def _sc_body(
    idx_hbm, tab_hbm, out_hbm, idx_v, tab_v, out_v,
    sem_t, sem_i0, sem_i1, sem_o0, sem_o1,
):
    c = lax.axis_index("c")
    s = lax.axis_index("s")
    wid = s * 2 + c
    base = wid * BPW
    cp_tab = pltpu.async_copy(tab_hbm, tab_v, sem_t)
    cp_i0 = pltpu.async_copy(
        idx_hbm.at[pl.ds(base, HALF)], idx_v.at[pl.ds(0, HALF)], sem_i0
    )
    cp_i1 = pltpu.async_copy(
        idx_hbm.at[pl.ds(base + HALF, HALF)],
        idx_v.at[pl.ds(HALF, HALF)],
        sem_i1,
    )
    iota = lax.iota(jnp.int32, L)
    two_iota = iota * 2
    col0 = jnp.zeros((L,), jnp.int32)
    col1 = jnp.ones((L,), jnp.int32)

    def compute(k):
        idx16 = idx_v[pl.ds(k * L, L)]
        g0 = plsc.load_gather(tab_v, [idx16, col0])
        g1 = plsc.load_gather(tab_v, [idx16, col1])
        plsc.store_scatter(out_v, [two_iota + k * 2 * L], g0)
        plsc.store_scatter(out_v, [two_iota + (k * 2 * L + 1)], g1)

    cp_tab.wait()
    cp_i0.wait()
    for k in range(HALF // L):
        compute(k)
    cp_o0 = pltpu.async_copy(
        out_v.at[pl.ds(0, HALF * EMBED_DIM)],
        out_hbm.at[pl.ds(base * EMBED_DIM, HALF * EMBED_DIM)],
        sem_o0,
    )
    cp_i1.wait()
    for k in range(HALF // L, BPW // L):
        compute(k)
    cp_o1 = pltpu.async_copy(
        out_v.at[pl.ds(HALF * EMBED_DIM, HALF * EMBED_DIM)],
        out_hbm.at[
            pl.ds(base * EMBED_DIM + HALF * EMBED_DIM, HALF * EMBED_DIM)
        ],
        sem_o1,
    )
    cp_o0.wait()
    cp_o1.wait()


def kernel(x, emb_table):
    xi = x.astype(jnp.int32)
    mesh = plsc.VectorSubcoreMesh(core_axis_name="c", subcore_axis_name="s")
    out_flat = pl.kernel(
        _sc_body,
        out_type=jax.ShapeDtypeStruct((BATCH * EMBED_DIM,), jnp.float32),
        mesh=mesh,
        compiler_params=pltpu.CompilerParams(needs_layout_passes=False),
        scratch_types=[
            pltpu.VMEM((BPW,), jnp.int32),
            pltpu.VMEM((3, EMBED_DIM), jnp.float32),
            pltpu.VMEM((OPW,), jnp.float32),
            pltpu.SemaphoreType.DMA,
            pltpu.SemaphoreType.DMA,
            pltpu.SemaphoreType.DMA,
            pltpu.SemaphoreType.DMA,
            pltpu.SemaphoreType.DMA,
        ],
    )(xi, emb_table)
    return out_flat.reshape(BATCH, EMBED_DIM)


# speedup vs baseline: 1.0107x; 1.0092x over previous
import jax
import jax.numpy as jnp
from jax import lax
from jax.experimental import pallas as pl
from jax.experimental.pallas import tpu as pltpu
from jax.experimental.pallas import tpu_sc as plsc

BATCH = 16384
EMBED_DIM = 2
NUM_WORKERS = 16
BPW = BATCH // NUM_WORKERS
OPW = BPW * EMBED_DIM
L = 16


HALF = BPW // 2


def _sc_body(
    idx_hbm, tab_hbm, out_hbm, idx_v, tab_v, out_v,
    sem_t, sem_i0, sem_i1, sem_o0, sem_o1,
):
    c = lax.axis_index("c")
    s = lax.axis_index("s")
    wid = s + c * 0
    base = wid * BPW
    cp_tab = pltpu.async_copy(tab_hbm, tab_v, sem_t)
    cp_i0 = pltpu.async_copy(
        idx_hbm.at[pl.ds(base, HALF)], idx_v.at[pl.ds(0, HALF)], sem_i0
    )
    cp_i1 = pltpu.async_copy(
        idx_hbm.at[pl.ds(base + HALF, HALF)],
        idx_v.at[pl.ds(HALF, HALF)],
        sem_i1,
    )
    iota = lax.iota(jnp.int32, L)
    two_iota = iota * 2
    col0 = jnp.zeros((L,), jnp.int32)
    col1 = jnp.ones((L,), jnp.int32)

    def compute(k):
        idx16 = idx_v[pl.ds(k * L, L)]
        g0 = plsc.load_gather(tab_v, [idx16, col0])
        g1 = plsc.load_gather(tab_v, [idx16, col1])
        plsc.store_scatter(out_v, [two_iota + k * 2 * L], g0)
        plsc.store_scatter(out_v, [two_iota + (k * 2 * L + 1)], g1)

    cp_tab.wait()
    cp_i0.wait()
    for k in range(HALF // L):
        compute(k)
    cp_o0 = pltpu.async_copy(
        out_v.at[pl.ds(0, HALF * EMBED_DIM)],
        out_hbm.at[pl.ds(base * EMBED_DIM, HALF * EMBED_DIM)],
        sem_o0,
    )
    cp_i1.wait()
    for k in range(HALF // L, BPW // L):
        compute(k)
    cp_o1 = pltpu.async_copy(
        out_v.at[pl.ds(HALF * EMBED_DIM, HALF * EMBED_DIM)],
        out_hbm.at[
            pl.ds(base * EMBED_DIM + HALF * EMBED_DIM, HALF * EMBED_DIM)
        ],
        sem_o1,
    )
    cp_o0.wait()
    cp_o1.wait()


def kernel(x, emb_table):
    xi = x.astype(jnp.int32)
    mesh = plsc.VectorSubcoreMesh(
        core_axis_name="c", subcore_axis_name="s", num_cores=1
    )
    out_flat = pl.kernel(
        _sc_body,
        out_type=jax.ShapeDtypeStruct((BATCH * EMBED_DIM,), jnp.float32),
        mesh=mesh,
        compiler_params=pltpu.CompilerParams(needs_layout_passes=False),
        scratch_types=[
            pltpu.VMEM((BPW,), jnp.int32),
            pltpu.VMEM((3, EMBED_DIM), jnp.float32),
            pltpu.VMEM((OPW,), jnp.float32),
            pltpu.SemaphoreType.DMA,
            pltpu.SemaphoreType.DMA,
            pltpu.SemaphoreType.DMA,
            pltpu.SemaphoreType.DMA,
            pltpu.SemaphoreType.DMA,
        ],
    )(xi, emb_table)
    return out_flat.reshape(BATCH, EMBED_DIM)


# speedup vs baseline: 1.0348x; 1.0239x over previous
import jax
import jax.numpy as jnp
from jax import lax
from jax.experimental import pallas as pl
from jax.experimental.pallas import tpu as pltpu
from jax.experimental.pallas import tpu_sc as plsc

BATCH = 16384
EMBED_DIM = 2
NUM_WORKERS = 16
BPW = BATCH // NUM_WORKERS
OPW = BPW * EMBED_DIM
L = 16
NCHUNK = 4
CHUNK = BPW // NCHUNK


def _sc_body(idx_hbm, tab_hbm, out_hbm, idx_v, tab_v, out_v, sems):
    s = lax.axis_index("s")
    base = s * BPW
    sem_t, sem_i, sem_o = sems
    cp_tab = pltpu.async_copy(tab_hbm, tab_v, sem_t)
    cp_in = [
        pltpu.async_copy(
            idx_hbm.at[pl.ds(base + q * CHUNK, CHUNK)],
            idx_v.at[pl.ds(q * CHUNK, CHUNK)],
            sem_i[q],
        )
        for q in range(NCHUNK)
    ]

    iota = lax.iota(jnp.int32, L)
    two_iota = iota * 2
    cp_tab.wait()
    zero = jnp.zeros((L,), jnp.int32)
    one = jnp.ones((L,), jnp.int32)
    two = jnp.full((L,), 2, jnp.int32)
    t00 = plsc.load_gather(tab_v, [zero, zero])
    t01 = plsc.load_gather(tab_v, [zero, one])
    t10 = plsc.load_gather(tab_v, [one, zero])
    t11 = plsc.load_gather(tab_v, [one, one])
    t20 = plsc.load_gather(tab_v, [two, zero])
    t21 = plsc.load_gather(tab_v, [two, one])

    cp_out = []
    for q in range(NCHUNK):
        cp_in[q].wait()
        for k in range(q * (CHUNK // L), (q + 1) * (CHUNK // L)):
            idx16 = idx_v[pl.ds(k * L, L)]
            is0 = idx16 == 0
            is1 = idx16 == 1
            g0 = jnp.where(is0, t00, jnp.where(is1, t10, t20))
            g1 = jnp.where(is0, t01, jnp.where(is1, t11, t21))
            plsc.store_scatter(out_v, [two_iota + k * 2 * L], g0)
            plsc.store_scatter(out_v, [two_iota + (k * 2 * L + 1)], g1)
        cp_out.append(
            pltpu.async_copy(
                out_v.at[pl.ds(q * CHUNK * EMBED_DIM, CHUNK * EMBED_DIM)],
                out_hbm.at[
                    pl.ds(
                        base * EMBED_DIM + q * CHUNK * EMBED_DIM,
                        CHUNK * EMBED_DIM,
                    )
                ],
                sem_o[q],
            )
        )
    for cp in cp_out:
        cp.wait()


def kernel(x, emb_table):
    xi = x.astype(jnp.int32)
    mesh = plsc.VectorSubcoreMesh(
        core_axis_name="c", subcore_axis_name="s", num_cores=1
    )
    out_flat = pl.kernel(
        _sc_body,
        out_type=jax.ShapeDtypeStruct((BATCH * EMBED_DIM,), jnp.float32),
        mesh=mesh,
        compiler_params=pltpu.CompilerParams(needs_layout_passes=False),
        scratch_types=[
            pltpu.VMEM((BPW,), jnp.int32),
            pltpu.VMEM((3, EMBED_DIM), jnp.float32),
            pltpu.VMEM((OPW,), jnp.float32),
            (
                pltpu.SemaphoreType.DMA,
                [pltpu.SemaphoreType.DMA] * NCHUNK,
                [pltpu.SemaphoreType.DMA] * NCHUNK,
            ),
        ],
    )(xi, emb_table)
    return out_flat.reshape(BATCH, EMBED_DIM)
